# speedup vs baseline: 3.9121x; 3.9121x over previous
from math import sqrt

import jax
import jax.numpy as jnp
from jax import lax
from jax.experimental import pallas as pl

N = 35
E = N * N
P = 48

_BN_SCALE = 1.0 / sqrt(1.0 + 0.001)


def _dot(a, b):
    return lax.dot_general(a, b, (((1,), (0,)), ((), ())),
                           preferred_element_type=jnp.float32,
                           precision=lax.Precision.HIGHEST)


def _dotT(a, b):
    return lax.dot_general(a, b, (((0,), (0,)), ((), ())),
                           preferred_element_type=jnp.float32,
                           precision=lax.Precision.HIGHEST)


def _sigmoid(z):
    return 1.0 / (1.0 + jnp.exp(-z))


def _body(src_ref, dst_ref, a_ref, x_ref,
          w1_ref, b1_ref, root1_ref, bias1_ref, bns1_ref, bnb1_ref,
          w2_ref, b2_ref, root2_ref, sc2_ref,
          w3_ref, b3_ref, root3_ref, bias3_ref, bns3_ref, bnb3_ref,
          out_ref):
    src = src_ref[:, :]
    dst = dst_ref[:, :]
    a = a_ref[:, :]
    cols = lax.broadcasted_iota(jnp.int32, (E, P), 1)
    oh_src = (cols == src).astype(jnp.float32)
    oh_dst = (cols == dst).astype(jnp.float32)

    xp = x_ref[:, :]
    xj = _dot(oh_src, xp)

    msg = jnp.zeros((E, P), jnp.float32)
    for i in range(N):
        w_i = jnp.maximum(a * w1_ref[i:i + 1, :] + b1_ref[i:i + 1, :], 0.0)
        msg = msg + xj[:, i:i + 1] * w_i
    s1 = _dotT(oh_dst, msg)
    cnt = _dotT(oh_dst, jnp.ones((E, 1), jnp.float32))
    inv_cnt = 1.0 / jnp.maximum(cnt, 1.0)
    pre1 = s1 * inv_cnt + _dot(xp, root1_ref[:, :]) + bias1_ref[0:1, :]
    x1 = _sigmoid(pre1 * bns1_ref[0:1, :] + bnb1_ref[0:1, :])

    w2 = jnp.maximum(a * w2_ref[0:1, :] + b2_ref[0:1, :], 0.0)
    xj2 = _dot(oh_src, x1)
    msg2 = jnp.sum(xj2 * w2, axis=1, keepdims=True)
    s2 = _dotT(oh_dst, msg2)
    pre2 = s2 * inv_cnt + _dot(x1, root2_ref[:, :]) + sc2_ref[0:1, 0:1]
    x2 = _sigmoid(pre2 * sc2_ref[0:1, 1:2] + sc2_ref[0:1, 2:3])

    w3 = jnp.maximum(a * w3_ref[0:1, :] + b3_ref[0:1, :], 0.0)
    xj3 = _dot(oh_src, x2)
    s3 = _dotT(oh_dst, xj3 * w3)
    pre3 = s3 * inv_cnt + x2 * root3_ref[0:1, :] + bias3_ref[0:1, :]
    x4 = _sigmoid(pre3 * bns3_ref[0:1, :] + bnb3_ref[0:1, :])

    x6 = (x4 + x1) * 0.5
    x6t = _dotT(x6, jnp.eye(P, dtype=jnp.float32))
    res = (x6 + x6t) * 0.5
    out_ref[:, :] = res[0:N, 0:N]


@jax.jit
def kernel(x, edge_index, edge_attr, c1_nnW, c1_nnb, c1_root, c1_bias,
           bn1_g, bn1_b, c2_nnW, c2_nnb, c2_root, c2_bias, bn2_g, bn2_b,
           c3_nnW, c3_nnb, c3_root, c3_bias, bn3_g, bn3_b):
    f32 = jnp.float32

    def pad2(m, r, c):
        m = m.astype(f32)
        return jnp.zeros((r, c), f32).at[:m.shape[0], :m.shape[1]].set(m)

    src = edge_index[0].reshape(E, 1).astype(jnp.int32)
    dst = edge_index[1].reshape(E, 1).astype(jnp.int32)

    xp = pad2(x, P, P)
    w1 = pad2(c1_nnW.reshape(N, N), N, P)
    b1 = pad2(c1_nnb.reshape(N, N), N, P)
    root1 = pad2(c1_root, P, P)
    bias1 = pad2(c1_bias.reshape(1, N), 1, P)
    bns1 = pad2((bn1_g * _BN_SCALE).reshape(1, N), 1, P)
    bnb1 = pad2(bn1_b.reshape(1, N), 1, P)

    w2 = pad2(c2_nnW.reshape(1, N), 1, P)
    b2 = pad2(c2_nnb.reshape(1, N), 1, P)
    root2 = pad2(c2_root, P, 1)
    sc2 = jnp.stack([c2_bias[0], bn2_g[0] * _BN_SCALE, bn2_b[0]]) \
             .astype(f32).reshape(1, 3)

    w3 = pad2(c3_nnW.reshape(1, N), 1, P)
    b3 = pad2(c3_nnb.reshape(1, N), 1, P)
    root3 = pad2(c3_root.reshape(1, N), 1, P)
    bias3 = pad2(c3_bias.reshape(1, N), 1, P)
    bns3 = pad2((bn3_g * _BN_SCALE).reshape(1, N), 1, P)
    bnb3 = pad2(bn3_b.reshape(1, N), 1, P)

    return pl.pallas_call(
        _body,
        out_shape=jax.ShapeDtypeStruct((N, N), f32),
    )(src, dst, edge_attr.astype(f32), xp,
      w1, b1, root1, bias1, bns1, bnb1,
      w2, b2, root2, sc2,
      w3, b3, root3, bias3, bns3, bnb3)
